# baseline (speedup 1.0000x reference)
import functools

import jax
import jax.numpy as jnp
from jax import lax
from jax.experimental import pallas as pl
from jax.experimental.pallas import tpu as pltpu
from jax.experimental.pallas import tpu_sc as plsc

NM = 2
NE = 8
TOPK = 2
DIN = 1024
DOUT = 1024
DH = 2048
NB = 2048
BLK = 128
NPAIR = NM * TOPK * NB
PPAD = NPAIR + NE * BLK
NG = PPAD // BLK
EPS = 2.220446049250313e-16
RB = 512


def _gate_dispatch(x, w_gate):
    def body(x_ref, wg_ref, g_out, pk_out, ps_out, be_out, act_out, loss_out):
        lg = jnp.dot(x_ref[...], wg_ref[...],
                     preferred_element_type=jnp.float32)
        col = lax.broadcasted_iota(jnp.int32, (NB, 16), 1)
        neg = jnp.float32(-jnp.inf)

        def top2(lo, hi):
            m = (col >= lo) & (col < hi)
            v = jnp.where(m, lg, neg)
            v1 = jnp.max(v, axis=1, keepdims=True)
            i1 = jnp.min(jnp.where(v == v1, col, 127), axis=1, keepdims=True)
            vm = jnp.where(col == i1, neg, v)
            v2 = jnp.max(vm, axis=1, keepdims=True)
            i2 = jnp.min(jnp.where(vm == v2, col, 127), axis=1, keepdims=True)
            ed = jnp.exp(v2 - v1)
            den = 1.0 + ed
            return i1, i2, 1.0 / den, ed / den

        i1a, i2a, g1a, g2a = top2(0, 8)
        i1b, i2b, g1b, g2b = top2(8, 16)

        ohs1 = (col == i1a) | (col == i1b)
        ohs2 = (col == i2a) | (col == i2b)
        gf1 = jnp.where(col < 8, g1a, g1b)
        gf2 = jnp.where(col < 8, g2a, g2b)
        imp = jnp.sum(jnp.where(ohs1, gf1, 0.0) + jnp.where(ohs2, gf2, 0.0),
                      axis=0, keepdims=True)
        ld = jnp.sum(
            jnp.where(ohs1 & (gf1 > 0.0), 1.0, 0.0)
            + jnp.where(ohs2 & (gf2 > 0.0), 1.0, 0.0),
            axis=0, keepdims=True)
        lane = lax.broadcasted_iota(jnp.int32, (1, 16), 1)
        m0 = jnp.where(lane < 8, 1.0, 0.0)
        m1 = jnp.where((lane >= 8) & (lane < 16), 1.0, 0.0)

        def cv(v, msk):
            mean = jnp.sum(v * msk, axis=1, keepdims=True) / 8.0
            var = jnp.sum(msk * (v - mean) ** 2, axis=1, keepdims=True) / 7.0
            return var / (mean * mean + 1e-08)

        loss_out[...] = cv(imp, m0) + cv(ld, m0) + cv(imp, m1) + cv(ld, m1)

        e0, e1 = i1a, i2a
        e2 = i1b & 7
        e3 = i2b & 7
        d2a = e2 == e0
        d2b = e2 == e1
        d3a = e3 == e0
        d3b = e3 == e1
        dup2 = d2a | d2b
        dup3 = d3a | d3b
        fcols = (e0, e1, jnp.where(dup2, NE, e2), jnp.where(dup3, NE, e3))

        r5 = lax.broadcasted_iota(jnp.int32, (RB, RB), 0)
        c5 = lax.broadcasted_iota(jnp.int32, (RB, RB), 1)
        ltri = jnp.where(c5 < r5, 1.0, 0.0)
        lane5 = lax.broadcasted_iota(jnp.int32, (RB, 16), 1)
        carry = jnp.zeros((1, 16), jnp.float32)
        rank_cols = []
        for s in range(4):
            fc = fcols[s]
            rblk = []
            for bb in range(NB // RB):
                fb = lax.slice(fc, (bb * RB, 0), ((bb + 1) * RB, 1))
                ohb = jnp.where(fb == lane5, 1.0, 0.0)
                pref = jnp.dot(ltri, ohb,
                               preferred_element_type=jnp.float32) + carry
                rblk.append(jnp.sum(jnp.where(fb == lane5, pref, 0.0),
                                    axis=1, keepdims=True))
                carry = carry + jnp.sum(ohb, axis=0, keepdims=True)
            rank_cols.append(jnp.concatenate(rblk, axis=0))

        cnt = carry
        nblk = jnp.floor((cnt + (BLK - 1)) * (1.0 / BLK))
        r16 = lax.broadcasted_iota(jnp.int32, (16, 16), 0)
        c16 = lax.broadcasted_iota(jnp.int32, (16, 16), 1)
        mincl = jnp.where((r16 <= c16) & (r16 < 8), 1.0, 0.0)
        blk_end = jnp.dot(nblk, mincl, preferred_element_type=jnp.float32)
        poff = (blk_end - nblk) * float(BLK)

        pos_cols = []
        for s in range(4):
            fc = fcols[s]
            lane_nb = lax.broadcasted_iota(jnp.int32, (NB, 16), 1)
            pofff = jnp.sum(jnp.where(fc == lane_nb, poff, 0.0),
                            axis=1, keepdims=True)
            pos_cols.append((pofff + rank_cols[s]).astype(jnp.int32))

        p0, p1, p2r, p3r = pos_cols
        p2 = jnp.where(d2a, p0, jnp.where(d2b, p1, p2r))
        p3 = jnp.where(d3a, p0, jnp.where(d3b, p1, p3r))
        pk_out[...] = jnp.concatenate([p0, p1, p2, p3], axis=1)
        ps_out[...] = jnp.concatenate(
            [p0, p1,
             jnp.where(dup2, PPAD, p2r), jnp.where(dup3, PPAD, p3r)], axis=1)
        g_out[...] = jnp.concatenate([g1a, g2a, g1b, g2b], axis=1)

        blk_end_i = blk_end.astype(jnp.int32)
        iota_g = lax.broadcasted_iota(jnp.int32, (NG, 16), 0)
        lane_g = lax.broadcasted_iota(jnp.int32, (NG, 16), 1)
        be = jnp.sum(jnp.where((blk_end_i <= iota_g) & (lane_g < 8), 1, 0),
                     axis=1, keepdims=True)
        be_out[...] = jnp.minimum(be, NE - 1)
        total = jnp.sum(jnp.where(lane == 7, blk_end_i, 0),
                        axis=1, keepdims=True)
        act_out[...] = jnp.where(
            lax.broadcasted_iota(jnp.int32, (NG, 1), 0) < total, 1, 0)

    wg16 = jnp.concatenate([w_gate[0], w_gate[1]], axis=1)
    return pl.pallas_call(
        body,
        grid=(1,),
        in_specs=[
            pl.BlockSpec((NB, DIN), lambda i: (0, 0)),
            pl.BlockSpec((DIN, 16), lambda i: (0, 0)),
        ],
        out_specs=[
            pl.BlockSpec((NB, 4), lambda i: (0, 0)),
            pl.BlockSpec((NB, 4), lambda i: (0, 0)),
            pl.BlockSpec((NB, 4), lambda i: (0, 0)),
            pl.BlockSpec((NG, 1), lambda i: (0, 0)),
            pl.BlockSpec((NG, 1), lambda i: (0, 0)),
            pl.BlockSpec((1, 1), lambda i: (0, 0)),
        ],
        out_shape=[
            jax.ShapeDtypeStruct((NB, 4), jnp.float32),
            jax.ShapeDtypeStruct((NB, 4), jnp.int32),
            jax.ShapeDtypeStruct((NB, 4), jnp.int32),
            jax.ShapeDtypeStruct((NG, 1), jnp.int32),
            jax.ShapeDtypeStruct((NG, 1), jnp.int32),
            jax.ShapeDtypeStruct((1, 1), jnp.float32),
        ],
    )(x, wg16)


def _sc_gather(table, idx, nrows, chunk):
    d = table.shape[1]
    nw = 32
    rows_pw = nrows // nw
    nchunk = rows_pw // chunk
    mesh = plsc.VectorSubcoreMesh(core_axis_name="c", subcore_axis_name="s")

    @functools.partial(
        pl.kernel,
        out_type=jax.ShapeDtypeStruct((nrows, d), jnp.float32),
        mesh=mesh,
        scratch_types=[
            pltpu.VMEM((rows_pw,), jnp.int32),
            pltpu.VMEM((chunk, d), jnp.float32),
            pltpu.VMEM((chunk, d), jnp.float32),
            pltpu.SemaphoreType.DMA,
            pltpu.SemaphoreType.DMA, pltpu.SemaphoreType.DMA,
            pltpu.SemaphoreType.DMA, pltpu.SemaphoreType.DMA,
        ],
    )
    def gather_k(table_hbm, idx_hbm, out_hbm, idx_v, buf0, buf1,
                 isem, g0, g1, w0, w1):
        wid = lax.axis_index("s") * 2 + lax.axis_index("c")
        base = wid * rows_pw
        pltpu.async_copy(idx_hbm.at[pl.ds(base, rows_pw)], idx_v, isem).wait()
        bufs = (buf0, buf1)
        gs = (g0, g1)
        ws = (w0, w1)
        gh = [None, None]
        wh = [None, None]
        for c in range(nchunk):
            b = c & 1
            if wh[b] is not None:
                wh[b].wait()
            gh[b] = pltpu.async_copy(
                table_hbm.at[idx_v.at[pl.ds(c * chunk, chunk)]], bufs[b], gs[b])
            if c >= 1:
                pb = (c - 1) & 1
                gh[pb].wait()
                wh[pb] = pltpu.async_copy(
                    bufs[pb], out_hbm.at[pl.ds(base + (c - 1) * chunk, chunk)],
                    ws[pb])
        lb = (nchunk - 1) & 1
        gh[lb].wait()
        wh[lb] = pltpu.async_copy(
            bufs[lb], out_hbm.at[pl.ds(base + (nchunk - 1) * chunk, chunk)],
            ws[lb])
        if nchunk >= 2:
            wh[1 - lb].wait()
        wh[lb].wait()

    return gather_k(table, idx)


def _expert_mlp(xs, fc1_w, fc1_b, fc2_w, fc2_b, be_ix, active):
    def body(be_ref, act_ref, xs_ref, w1_ref, b1_ref, w2_ref, b2_ref, ys_ref):
        g = pl.program_id(0)

        @pl.when(act_ref[g] == 1)
        def _():
            h = jnp.dot(xs_ref[...], w1_ref[0],
                        preferred_element_type=jnp.float32) + b1_ref[0]
            h = jnp.maximum(h, 0.0)
            o = jnp.dot(h, w2_ref[0],
                        preferred_element_type=jnp.float32) + b2_ref[0]
            ys_ref[...] = jnp.exp(o)

    grid_spec = pltpu.PrefetchScalarGridSpec(
        num_scalar_prefetch=2,
        grid=(NG,),
        in_specs=[
            pl.BlockSpec((BLK, DIN), lambda g, be, act: (g, 0)),
            pl.BlockSpec((1, DIN, DH), lambda g, be, act: (be[g], 0, 0)),
            pl.BlockSpec((1, 1, DH), lambda g, be, act: (be[g], 0, 0)),
            pl.BlockSpec((1, DH, DOUT), lambda g, be, act: (be[g], 0, 0)),
            pl.BlockSpec((1, 1, DOUT), lambda g, be, act: (be[g], 0, 0)),
        ],
        out_specs=pl.BlockSpec((BLK, DOUT), lambda g, be, act: (g, 0)),
    )
    return pl.pallas_call(
        body,
        grid_spec=grid_spec,
        out_shape=jax.ShapeDtypeStruct((PPAD, DOUT), jnp.float32),
    )(be_ix, active, xs, fc1_w, fc1_b.reshape(NE, 1, DH),
      fc2_w, fc2_b.reshape(NE, 1, DOUT))


def _combine_log(yk4, gsel):
    rows = 256
    nchunk = NB // rows

    def body(yk_ref, g_ref, out_ref):
        m = pl.program_id(0)
        y1 = yk_ref[:, 0, 0, :]
        y2 = yk_ref[:, 0, 1, :]
        gch = g_ref[...]
        lane4 = lax.broadcasted_iota(jnp.int32, (rows, 4), 1)
        a1 = jnp.sum(jnp.where(lane4 == 2 * m, gch, 0.0),
                     axis=1, keepdims=True)
        a2 = jnp.sum(jnp.where(lane4 == 2 * m + 1, gch, 0.0),
                     axis=1, keepdims=True)
        comb = a1 * y1 + a2 * y2
        comb = jnp.where(comb == 0.0, EPS, comb)
        out_ref[0] = jnp.log(comb)

    return pl.pallas_call(
        body,
        grid=(NM, nchunk),
        in_specs=[
            pl.BlockSpec((rows, 1, TOPK, DOUT), lambda m, c: (c, m, 0, 0)),
            pl.BlockSpec((rows, 4), lambda m, c: (c, 0)),
        ],
        out_specs=pl.BlockSpec((1, rows, DOUT), lambda m, c: (m, c, 0)),
        out_shape=jax.ShapeDtypeStruct((NM, NB, DOUT), jnp.float32),
    )(yk4, gsel)


def kernel(x, train, w_gate, fc1_w, fc1_b, fc2_w, fc2_b, loss_coef):
    gsel, pk, pos_scatter, be_ix, active, loss = _gate_dispatch(x, w_gate)
    b_flat = jnp.arange(NPAIR, dtype=jnp.int32) >> 2
    pad_tok = jnp.arange(PPAD, dtype=jnp.int32) & (NB - 1)
    tok_sorted = pad_tok.at[pos_scatter.reshape(-1)].set(b_flat, mode='drop')
    xs = _sc_gather(x, tok_sorted, PPAD, 48)
    ys = _expert_mlp(xs, fc1_w, fc1_b, fc2_w, fc2_b,
                     be_ix.reshape(NG), active.reshape(NG))
    yk = _sc_gather(ys, pk.reshape(-1), NPAIR, 32)
    all_y = _combine_log(yk.reshape(NB, NM, TOPK, DOUT), gsel)
    return (all_y, loss[0, 0] * loss_coef)


# speedup vs baseline: 1.1478x; 1.1478x over previous
import functools

import jax
import jax.numpy as jnp
from jax import lax
from jax.experimental import pallas as pl
from jax.experimental.pallas import tpu as pltpu
from jax.experimental.pallas import tpu_sc as plsc

NM = 2
NE = 8
TOPK = 2
DIN = 1024
DOUT = 1024
DH = 2048
NB = 2048
BLK = 128
NPAIR = NM * TOPK * NB
PPAD = NPAIR + NE * BLK
NG = PPAD // BLK
EPS = 2.220446049250313e-16


def _cv(v):
    m = jnp.mean(v)
    return jnp.var(v, ddof=1) / (m * m + 1e-08)


def _routing(x, w_gate, loss_coef):
    logits = jnp.einsum('bd,mde->mbe', x, w_gate,
                        preferred_element_type=jnp.float32)
    i1 = jnp.argmax(logits, axis=-1)
    v1 = jnp.max(logits, axis=-1)
    arange_e = jnp.arange(NE, dtype=jnp.int32)
    oh1 = i1[..., None] == arange_e
    masked = jnp.where(oh1, -jnp.inf, logits)
    i2 = jnp.argmax(masked, axis=-1)
    v2 = jnp.max(masked, axis=-1)
    oh2 = i2[..., None] == arange_e
    ed = jnp.exp(v2 - v1)
    denom = 1.0 + ed
    g1 = 1.0 / denom
    g2 = ed / denom

    importance = (oh1 * g1[..., None] + oh2 * g2[..., None]).sum(axis=1)
    load = (oh1 * (g1 > 0.0)[..., None] + oh2 * (g2 > 0.0)[..., None]
            ).sum(axis=1).astype(jnp.float32)
    loss = (_cv(importance[0]) + _cv(load[0])
            + _cv(importance[1]) + _cv(load[1])) * loss_coef

    e_all = jnp.stack([i1[0], i2[0], i1[1], i2[1]]).astype(jnp.int32)
    dup2 = (e_all[2] == e_all[0]) | (e_all[2] == e_all[1])
    dup3 = (e_all[3] == e_all[0]) | (e_all[3] == e_all[1])
    no_dup = jnp.zeros_like(dup2)
    dup = jnp.stack([no_dup, no_dup, dup2, dup3])

    f = jnp.where(dup, NE, e_all).reshape(-1)
    ohT = (jnp.arange(NE + 1, dtype=jnp.int32)[:, None] == f[None, :]
           ).astype(jnp.int32)
    cumT = jnp.cumsum(ohT, axis=1)
    cnt = cumT[:NE, -1]
    rank = (ohT * cumT).sum(axis=0) - 1
    nblk = (cnt + BLK - 1) // BLK
    blk_end = jnp.cumsum(nblk)
    poff = jnp.concatenate([(blk_end - nblk) * BLK,
                            jnp.zeros((1,), jnp.int32)])
    pos = poff[f] + rank
    total_blocks = blk_end[-1]

    pos_s = pos.reshape(NM * TOPK, NB)
    pos2 = jnp.where(e_all[2] == e_all[0], pos_s[0],
                     jnp.where(e_all[2] == e_all[1], pos_s[1], pos_s[2]))
    pos3 = jnp.where(e_all[3] == e_all[0], pos_s[0],
                     jnp.where(e_all[3] == e_all[1], pos_s[1], pos_s[3]))
    pk = jnp.stack([pos_s[0], pos_s[1], pos2, pos3]).reshape(-1)

    be = jnp.searchsorted(blk_end.astype(jnp.int32),
                          jnp.arange(NG, dtype=jnp.int32), side='right')
    be_ix = jnp.minimum(be, NE - 1).astype(jnp.int32)
    active = (jnp.arange(NG) < total_blocks).astype(jnp.int32)

    b_flat = jnp.tile(jnp.arange(NB, dtype=jnp.int32), NM * TOPK)
    pos_scatter = jnp.where(dup.reshape(-1), PPAD, pos)
    pad_tok = jnp.arange(PPAD, dtype=jnp.int32) % NB
    tok_sorted = pad_tok.at[pos_scatter].set(b_flat, mode='drop')
    return g1, g2, loss, tok_sorted, pk.astype(jnp.int32), be_ix, active


def _sc_gather(table, idx, nrows, chunk):
    d = table.shape[1]
    nw = 32
    rows_pw = nrows // nw
    nchunk = rows_pw // chunk
    mesh = plsc.VectorSubcoreMesh(core_axis_name="c", subcore_axis_name="s")

    @functools.partial(
        pl.kernel,
        out_type=jax.ShapeDtypeStruct((nrows, d), jnp.float32),
        mesh=mesh,
        scratch_types=[
            pltpu.VMEM((rows_pw,), jnp.int32),
            pltpu.VMEM((chunk, d), jnp.float32),
            pltpu.VMEM((chunk, d), jnp.float32),
            pltpu.SemaphoreType.DMA,
            pltpu.SemaphoreType.DMA, pltpu.SemaphoreType.DMA,
            pltpu.SemaphoreType.DMA, pltpu.SemaphoreType.DMA,
        ],
    )
    def gather_k(table_hbm, idx_hbm, out_hbm, idx_v, buf0, buf1,
                 isem, g0, g1, w0, w1):
        wid = lax.axis_index("s") * 2 + lax.axis_index("c")
        base = wid * rows_pw
        pltpu.async_copy(idx_hbm.at[pl.ds(base, rows_pw)], idx_v, isem).wait()
        bufs = (buf0, buf1)
        gs = (g0, g1)
        ws = (w0, w1)
        gh = [None, None]
        wh = [None, None]
        for c in range(nchunk):
            b = c & 1
            if wh[b] is not None:
                wh[b].wait()
            gh[b] = pltpu.async_copy(
                table_hbm.at[idx_v.at[pl.ds(c * chunk, chunk)]], bufs[b], gs[b])
            if c >= 1:
                pb = (c - 1) & 1
                gh[pb].wait()
                wh[pb] = pltpu.async_copy(
                    bufs[pb], out_hbm.at[pl.ds(base + (c - 1) * chunk, chunk)],
                    ws[pb])
        lb = (nchunk - 1) & 1
        gh[lb].wait()
        wh[lb] = pltpu.async_copy(
            bufs[lb], out_hbm.at[pl.ds(base + (nchunk - 1) * chunk, chunk)],
            ws[lb])
        if nchunk >= 2:
            wh[1 - lb].wait()
        wh[lb].wait()

    return gather_k(table, idx)


def _expert_mlp(xs, fc1_w, fc1_b, fc2_w, fc2_b, be_ix, active):
    def body(be_ref, act_ref, xs_ref, w1_ref, b1_ref, w2_ref, b2_ref, ys_ref):
        g = pl.program_id(0)

        @pl.when(act_ref[g] == 1)
        def _():
            h = jnp.dot(xs_ref[...], w1_ref[0],
                        preferred_element_type=jnp.float32) + b1_ref[0]
            h = jnp.maximum(h, 0.0)
            o = jnp.dot(h, w2_ref[0],
                        preferred_element_type=jnp.float32) + b2_ref[0]
            ys_ref[...] = jnp.exp(o)

    grid_spec = pltpu.PrefetchScalarGridSpec(
        num_scalar_prefetch=2,
        grid=(NG,),
        in_specs=[
            pl.BlockSpec((BLK, DIN), lambda g, be, act: (g, 0)),
            pl.BlockSpec((1, DIN, DH), lambda g, be, act: (be[g], 0, 0)),
            pl.BlockSpec((1, 1, DH), lambda g, be, act: (be[g], 0, 0)),
            pl.BlockSpec((1, DH, DOUT), lambda g, be, act: (be[g], 0, 0)),
            pl.BlockSpec((1, 1, DOUT), lambda g, be, act: (be[g], 0, 0)),
        ],
        out_specs=pl.BlockSpec((BLK, DOUT), lambda g, be, act: (g, 0)),
    )
    return pl.pallas_call(
        body,
        grid_spec=grid_spec,
        out_shape=jax.ShapeDtypeStruct((PPAD, DOUT), jnp.float32),
    )(be_ix, active, xs, fc1_w, fc1_b.reshape(NE, 1, DH),
      fc2_w, fc2_b.reshape(NE, 1, DOUT))


def _combine_log(yk, g1, g2):
    rows = 256
    nchunk = NB // rows

    def body(yk_ref, g1_ref, g2_ref, out_ref):
        m = pl.program_id(0)
        c = pl.program_id(1)
        a1 = g1_ref[m, pl.ds(c * rows, rows)][:, None]
        a2 = g2_ref[m, pl.ds(c * rows, rows)][:, None]
        comb = a1 * yk_ref[0, 0] + a2 * yk_ref[0, 1]
        comb = jnp.where(comb == 0.0, EPS, comb)
        out_ref[0] = jnp.log(comb)

    return pl.pallas_call(
        body,
        grid=(NM, nchunk),
        in_specs=[
            pl.BlockSpec((1, 2, rows, DOUT), lambda m, c: (m, 0, c, 0)),
            pl.BlockSpec((NM, NB), lambda m, c: (0, 0)),
            pl.BlockSpec((NM, NB), lambda m, c: (0, 0)),
        ],
        out_specs=pl.BlockSpec((1, rows, DOUT), lambda m, c: (m, c, 0)),
        out_shape=jax.ShapeDtypeStruct((NM, NB, DOUT), jnp.float32),
    )(yk, g1, g2)


def kernel(x, train, w_gate, fc1_w, fc1_b, fc2_w, fc2_b, loss_coef):
    g1, g2, loss, tok_sorted, pk, be_ix, active = _routing(x, w_gate, loss_coef)
    xs = _sc_gather(x, tok_sorted, PPAD, 48)
    ys = _expert_mlp(xs, fc1_w, fc1_b, fc2_w, fc2_b, be_ix, active)
    yk = _sc_gather(ys, pk, NPAIR, 32)
    all_y = _combine_log(yk.reshape(NM, TOPK, NB, DOUT), g1, g2)
    return (all_y, loss)
